# baseline (speedup 1.0000x reference)
import jax
import jax.numpy as jnp
from jax import lax
from jax.experimental import pallas as pl
from jax.experimental.pallas import tpu as pltpu
from jax.experimental.pallas import tpu_sc as plsc

N = 10000
E = 320000
DF = 128
DH = 64
NC = 2
NS = 16
NW = NC * NS
EPW = E // NW
K = 80
NCH = EPW // K
RPT = 624
REM = N - NS * RPT
NB = 5
PF = 4
NP = NCH // NB
BR = 2000
GRID = N // BR

_MESH = plsc.VectorSubcoreMesh(
    core_axis_name="c", subcore_axis_name="s", num_cores=NC, num_subcores=NS
)


def _segsum_body(msgs, edges, zin, out, acc, idx_s, idx_d, *rest):
    c = lax.axis_index("c")
    s = lax.axis_index("s")
    wid = s * NC + c
    bufs = rest[:NB]
    gsems = rest[NB:2 * NB]
    ssems = rest[2 * NB:3 * NB]
    pltpu.sync_copy(zin, acc.at[pl.ds(s * RPT, RPT)])

    @pl.when(s == 0)
    def _():
        pltpu.sync_copy(zin.at[pl.ds(0, REM)], acc.at[pl.ds(NS * RPT, REM)])

    pltpu.sync_copy(edges.at[0, wid], idx_s)
    pltpu.sync_copy(edges.at[1, wid], idx_d)
    plsc.subcore_barrier()

    for b in range(PF):
        pltpu.async_copy(msgs.at[idx_s.at[b]], bufs[b], gsems[b])

    def step(p, carry):
        for b in range(NB):
            j = NB * p + b
            pltpu.make_async_copy(msgs.at[idx_s.at[j]], bufs[b], gsems[b]).wait()
            pltpu.async_copy(bufs[b], acc.at[idx_d.at[j]], ssems[b], add=True)
            nb2 = (b + PF) % NB
            jn = j + PF

            @pl.when(jnp.logical_and(jn >= NB, jn < NCH))
            def _():
                pltpu.make_async_copy(bufs[nb2], acc.at[idx_d.at[jn - NB]],
                                      ssems[nb2]).wait()

            @pl.when(jn < NCH)
            def _():
                pltpu.async_copy(msgs.at[idx_s.at[jn]], bufs[nb2], gsems[nb2])

        return carry

    lax.fori_loop(0, NP, step, 0)
    for b in range(NB):
        pltpu.make_async_copy(bufs[b], acc.at[idx_d.at[NCH - NB + b]],
                              ssems[b]).wait()
    plsc.subcore_barrier()
    pltpu.sync_copy(acc.at[pl.ds(s * RPT, RPT)], out.at[c, pl.ds(s * RPT, RPT)])

    @pl.when(s == 0)
    def _():
        pltpu.sync_copy(acc.at[pl.ds(NS * RPT, REM)],
                        out.at[c, pl.ds(NS * RPT, REM)])


_segsum = pl.kernel(
    _segsum_body,
    out_type=jax.ShapeDtypeStruct((NC, N, DH), jnp.float32),
    mesh=_MESH,
    scratch_types=[
        pltpu.VMEM_SHARED((N, DH), jnp.float32),
        pltpu.VMEM((NCH, K), jnp.int32),
        pltpu.VMEM((NCH, K), jnp.int32),
    ] + [pltpu.VMEM((K, DH), jnp.float32)] * NB
      + [pltpu.SemaphoreType.DMA] * (2 * NB),
    compiler_params=pltpu.CompilerParams(use_tc_tiling_on_sc=False),
)


def _deg_body(edges, zdeg, out, deg_o, deg_i, idx_s, idx_d, ones_b,
              dsem_o, dsem_i):
    c = lax.axis_index("c")
    s = lax.axis_index("s")
    wid = s * NC + c
    pltpu.sync_copy(zdeg, deg_o.at[pl.ds(s * RPT, RPT)])
    pltpu.sync_copy(zdeg, deg_i.at[pl.ds(s * RPT, RPT)])

    @pl.when(s == 0)
    def _():
        pltpu.sync_copy(zdeg.at[pl.ds(0, REM)], deg_o.at[pl.ds(NS * RPT, REM)])
        pltpu.sync_copy(zdeg.at[pl.ds(0, REM)], deg_i.at[pl.ds(NS * RPT, REM)])

    pltpu.sync_copy(edges.at[0, wid], idx_s)
    pltpu.sync_copy(edges.at[1, wid], idx_d)
    for t in range(K // 16):
        ones_b[pl.ds(t * 16, 16)] = jnp.ones((16,), jnp.float32)
    plsc.subcore_barrier()

    DPF = 4

    def chunk(j, carry):
        @pl.when(j >= DPF)
        def _():
            pltpu.make_async_copy(ones_b, deg_o.at[idx_s.at[j - DPF]],
                                  dsem_o).wait()
            pltpu.make_async_copy(ones_b, deg_i.at[idx_d.at[j - DPF]],
                                  dsem_i).wait()

        pltpu.async_copy(ones_b, deg_o.at[idx_s.at[j]], dsem_o, add=True)
        pltpu.async_copy(ones_b, deg_i.at[idx_d.at[j]], dsem_i, add=True)
        return carry

    lax.fori_loop(0, NCH, chunk, 0)
    for t in range(DPF):
        pltpu.make_async_copy(ones_b, deg_o.at[idx_s.at[NCH - DPF + t]],
                              dsem_o).wait()
        pltpu.make_async_copy(ones_b, deg_i.at[idx_d.at[NCH - DPF + t]],
                              dsem_i).wait()
    plsc.subcore_barrier()
    @pl.when(s < GRID)
    def _():
        pltpu.sync_copy(deg_o.at[pl.ds(s * BR, BR)], out.at[s, 2 * c])
        pltpu.sync_copy(deg_i.at[pl.ds(s * BR, BR)], out.at[s, 2 * c + 1])


_deg = pl.kernel(
    _deg_body,
    out_type=jax.ShapeDtypeStruct((GRID, 4, BR), jnp.float32),
    mesh=_MESH,
    scratch_types=[
        pltpu.VMEM_SHARED((N,), jnp.float32),
        pltpu.VMEM_SHARED((N,), jnp.float32),
        pltpu.VMEM((NCH, K), jnp.int32),
        pltpu.VMEM((NCH, K), jnp.int32),
        pltpu.VMEM((K,), jnp.float32),
        pltpu.SemaphoreType.DMA,
        pltpu.SemaphoreType.DMA,
    ],
    compiler_params=pltpu.CompilerParams(use_tc_tiling_on_sc=False),
)


def _col(v):
    return lax.dot_general(v, jnp.ones((1, 1), jnp.float32),
                           (((0,), (0,)), ((), ())),
                           preferred_element_type=jnp.float32,
                           precision=lax.Precision.HIGHEST)


def _norm_col(dp_ref, which):
    cols = []
    for g in range(GRID):
        r = dp_ref[g]
        dego = r[0:1, :] + r[2:3, :]
        degi = r[1:2, :] + r[3:4, :]
        if which == 0:
            v = lax.rsqrt(jnp.maximum(dego, 1.0))
        elif which == 1:
            v = lax.rsqrt(jnp.maximum(degi, 1.0))
        else:
            v = (lax.rsqrt(jnp.maximum(dego, 1.0))
                 * lax.rsqrt(jnp.maximum(degi, 1.0)))
        cols.append(_col(v))
    return jnp.concatenate(cols, axis=0)


def _tc_a_body(x_ref, w1_ref, dp_ref, h1_ref):
    ns = _norm_col(dp_ref, 0)
    h = jnp.dot(x_ref[...], w1_ref[...],
                preferred_element_type=jnp.float32,
                precision=lax.Precision.HIGHEST)
    h1_ref[...] = h * ns


_tc_a = pl.pallas_call(
    _tc_a_body,
    out_shape=jax.ShapeDtypeStruct((N, DH), jnp.float32),
)


def _tc_b_body(p_ref, dp_ref, h2_ref):
    nn = _norm_col(dp_ref, 2)
    h2_ref[...] = jnp.maximum(p_ref[0] + p_ref[1], 0.0) * nn


_tc_b = pl.pallas_call(
    _tc_b_body,
    out_shape=jax.ShapeDtypeStruct((N, DH), jnp.float32),
)


def _tc_c_body(p_ref, w2_ref, dp_ref, out_ref):
    nd = _norm_col(dp_ref, 1)
    agg = p_ref[0] + p_ref[1]
    out_ref[...] = jnp.dot(agg, w2_ref[...],
                           preferred_element_type=jnp.float32,
                           precision=lax.Precision.HIGHEST) * nd


_tc_c = pl.pallas_call(
    _tc_c_body,
    out_shape=jax.ShapeDtypeStruct((N, DF), jnp.float32),
)


def kernel(features, edge_index, W1, W2):
    edges = edge_index.astype(jnp.int32).reshape(2, NW, NCH, K)
    zin = jnp.zeros((RPT, DH), jnp.float32)
    zdeg = jnp.zeros((RPT,), jnp.float32)
    degp = _deg(edges, zdeg)
    h1 = _tc_a(features, W1, degp)
    p1 = _segsum(h1, edges, zin)
    h2 = _tc_b(p1, degp)
    p2 = _segsum(h2, edges, zin)
    return _tc_c(p2, W2, degp)


# speedup vs baseline: 1.1175x; 1.1175x over previous
import jax
import jax.numpy as jnp
from jax import lax
from jax.experimental import pallas as pl
from jax.experimental.pallas import tpu as pltpu
from jax.experimental.pallas import tpu_sc as plsc

N = 10000
E = 320000
DF = 128
DH = 64
NC = 2
NS = 16
NW = NC * NS
EPW = E // NW
K = 80
NCH = EPW // K
RPT = 624
REM = N - NS * RPT
NB = 5
PF = 4
NP = NCH // NB
BR = 2000
GRID = N // BR

_MESH = plsc.VectorSubcoreMesh(
    core_axis_name="c", subcore_axis_name="s", num_cores=NC, num_subcores=NS
)


def _segsum_body(msgs, edges, zin, out, acc, idx_s, idx_d, *rest):
    c = lax.axis_index("c")
    s = lax.axis_index("s")
    wid = s * NC + c
    bufs = rest[:NB]
    gsems = rest[NB:2 * NB]
    ssems = rest[2 * NB:3 * NB]
    pltpu.sync_copy(zin, acc.at[pl.ds(s * RPT, RPT)])

    @pl.when(s == 0)
    def _():
        pltpu.sync_copy(zin.at[pl.ds(0, REM)], acc.at[pl.ds(NS * RPT, REM)])

    pltpu.sync_copy(edges.at[0, wid], idx_s)
    pltpu.sync_copy(edges.at[1, wid], idx_d)
    plsc.subcore_barrier()

    for b in range(PF):
        pltpu.async_copy(msgs.at[idx_s.at[b]], bufs[b], gsems[b])

    def step(p, carry):
        for b in range(NB):
            j = NB * p + b
            pltpu.make_async_copy(msgs.at[idx_s.at[j]], bufs[b], gsems[b]).wait()
            pltpu.async_copy(bufs[b], acc.at[idx_d.at[j]], ssems[b], add=True)
            nb2 = (b + PF) % NB
            jn = j + PF

            @pl.when(jnp.logical_and(jn >= NB, jn < NCH))
            def _():
                pltpu.make_async_copy(bufs[nb2], acc.at[idx_d.at[jn - NB]],
                                      ssems[nb2]).wait()

            @pl.when(jn < NCH)
            def _():
                pltpu.async_copy(msgs.at[idx_s.at[jn]], bufs[nb2], gsems[nb2])

        return carry

    lax.fori_loop(0, NP, step, 0)
    for b in range(NB):
        pltpu.make_async_copy(bufs[b], acc.at[idx_d.at[NCH - NB + b]],
                              ssems[b]).wait()
    plsc.subcore_barrier()
    pltpu.sync_copy(acc.at[pl.ds(s * RPT, RPT)],
                    out.at[pl.ds(s * RPT, RPT), pl.ds(DH * c, DH)])

    @pl.when(s == 0)
    def _():
        pltpu.sync_copy(acc.at[pl.ds(NS * RPT, REM)],
                        out.at[pl.ds(NS * RPT, REM), pl.ds(DH * c, DH)])


_segsum = pl.kernel(
    _segsum_body,
    out_type=jax.ShapeDtypeStruct((N, DF), jnp.float32),
    mesh=_MESH,
    scratch_types=[
        pltpu.VMEM_SHARED((N, DH), jnp.float32),
        pltpu.VMEM((NCH, K), jnp.int32),
        pltpu.VMEM((NCH, K), jnp.int32),
    ] + [pltpu.VMEM((K, DH), jnp.float32)] * NB
      + [pltpu.SemaphoreType.DMA] * (2 * NB),
    compiler_params=pltpu.CompilerParams(use_tc_tiling_on_sc=False),
)


def _deg_body(edges, zdeg, out, deg_o, deg_i, idx_s, idx_d, ones_b,
              dsem_o, dsem_i):
    c = lax.axis_index("c")
    s = lax.axis_index("s")
    wid = s * NC + c
    pltpu.sync_copy(zdeg, deg_o.at[pl.ds(s * RPT, RPT)])
    pltpu.sync_copy(zdeg, deg_i.at[pl.ds(s * RPT, RPT)])

    @pl.when(s == 0)
    def _():
        pltpu.sync_copy(zdeg.at[pl.ds(0, REM)], deg_o.at[pl.ds(NS * RPT, REM)])
        pltpu.sync_copy(zdeg.at[pl.ds(0, REM)], deg_i.at[pl.ds(NS * RPT, REM)])

    pltpu.sync_copy(edges.at[0, wid], idx_s)
    pltpu.sync_copy(edges.at[1, wid], idx_d)
    for t in range(K // 16):
        ones_b[pl.ds(t * 16, 16)] = jnp.ones((16,), jnp.float32)
    plsc.subcore_barrier()

    DPF = 4

    def chunk(j, carry):
        @pl.when(j >= DPF)
        def _():
            pltpu.make_async_copy(ones_b, deg_o.at[idx_s.at[j - DPF]],
                                  dsem_o).wait()
            pltpu.make_async_copy(ones_b, deg_i.at[idx_d.at[j - DPF]],
                                  dsem_i).wait()

        pltpu.async_copy(ones_b, deg_o.at[idx_s.at[j]], dsem_o, add=True)
        pltpu.async_copy(ones_b, deg_i.at[idx_d.at[j]], dsem_i, add=True)
        return carry

    lax.fori_loop(0, NCH, chunk, 0)
    for t in range(DPF):
        pltpu.make_async_copy(ones_b, deg_o.at[idx_s.at[NCH - DPF + t]],
                              dsem_o).wait()
        pltpu.make_async_copy(ones_b, deg_i.at[idx_d.at[NCH - DPF + t]],
                              dsem_i).wait()
    plsc.subcore_barrier()
    @pl.when(s < GRID)
    def _():
        pltpu.sync_copy(deg_o.at[pl.ds(s * BR, BR)], out.at[s, 2 * c])
        pltpu.sync_copy(deg_i.at[pl.ds(s * BR, BR)], out.at[s, 2 * c + 1])


_deg = pl.kernel(
    _deg_body,
    out_type=jax.ShapeDtypeStruct((GRID, 4, BR), jnp.float32),
    mesh=_MESH,
    scratch_types=[
        pltpu.VMEM_SHARED((N,), jnp.float32),
        pltpu.VMEM_SHARED((N,), jnp.float32),
        pltpu.VMEM((NCH, K), jnp.int32),
        pltpu.VMEM((NCH, K), jnp.int32),
        pltpu.VMEM((K,), jnp.float32),
        pltpu.SemaphoreType.DMA,
        pltpu.SemaphoreType.DMA,
    ],
    compiler_params=pltpu.CompilerParams(use_tc_tiling_on_sc=False),
)


def _col(v):
    return lax.dot_general(v, jnp.ones((1, 1), jnp.float32),
                           (((0,), (0,)), ((), ())),
                           preferred_element_type=jnp.float32,
                           precision=lax.Precision.HIGHEST)


def _deg_rows(dp_ref):
    r = dp_ref[0]
    return r[0:1, :] + r[2:3, :], r[1:2, :] + r[3:4, :]


_DP_SPEC = pl.BlockSpec((1, 4, BR), lambda i: (i, 0, 0))


def _tc_a_body(x_ref, w1_ref, dp_ref, h1_ref):
    dego, _ = _deg_rows(dp_ref)
    ns = _col(lax.rsqrt(jnp.maximum(dego, 1.0)))
    h = jnp.dot(x_ref[...], w1_ref[...],
                preferred_element_type=jnp.float32,
                precision=lax.Precision.HIGHEST)
    h1_ref[...] = h * ns


_tc_a = pl.pallas_call(
    _tc_a_body,
    grid=(GRID,),
    in_specs=[
        pl.BlockSpec((BR, DF), lambda i: (i, 0)),
        pl.BlockSpec((DF, DH), lambda i: (0, 0)),
        _DP_SPEC,
    ],
    out_specs=pl.BlockSpec((BR, DH), lambda i: (i, 0)),
    out_shape=jax.ShapeDtypeStruct((N, DH), jnp.float32),
)


def _tc_b_body(p_ref, dp_ref, h2_ref):
    dego, degi = _deg_rows(dp_ref)
    nn = _col(lax.rsqrt(jnp.maximum(dego, 1.0))
              * lax.rsqrt(jnp.maximum(degi, 1.0)))
    agg = p_ref[:, 0:DH] + p_ref[:, DH:DF]
    h2_ref[...] = jnp.maximum(agg, 0.0) * nn


_tc_b = pl.pallas_call(
    _tc_b_body,
    grid=(GRID,),
    in_specs=[
        pl.BlockSpec((BR, DF), lambda i: (i, 0)),
        _DP_SPEC,
    ],
    out_specs=pl.BlockSpec((BR, DH), lambda i: (i, 0)),
    out_shape=jax.ShapeDtypeStruct((N, DH), jnp.float32),
)


def _tc_c_body(p_ref, w2_ref, dp_ref, out_ref):
    _, degi = _deg_rows(dp_ref)
    nd = _col(lax.rsqrt(jnp.maximum(degi, 1.0)))
    agg = p_ref[:, 0:DH] + p_ref[:, DH:DF]
    out_ref[...] = jnp.dot(agg, w2_ref[...],
                           preferred_element_type=jnp.float32,
                           precision=lax.Precision.HIGHEST) * nd


_tc_c = pl.pallas_call(
    _tc_c_body,
    grid=(GRID,),
    in_specs=[
        pl.BlockSpec((BR, DF), lambda i: (i, 0)),
        pl.BlockSpec((DH, DF), lambda i: (0, 0)),
        _DP_SPEC,
    ],
    out_specs=pl.BlockSpec((BR, DF), lambda i: (i, 0)),
    out_shape=jax.ShapeDtypeStruct((N, DF), jnp.float32),
)


def kernel(features, edge_index, W1, W2):
    edges = edge_index.astype(jnp.int32).reshape(2, NW, NCH, K)
    zin = jnp.zeros((RPT, DH), jnp.float32)
    zdeg = jnp.zeros((RPT,), jnp.float32)
    degp = _deg(edges, zdeg)
    h1 = _tc_a(features, W1, degp)
    p1 = _segsum(h1, edges, zin)
    h2 = _tc_b(p1, degp)
    p2 = _segsum(h2, edges, zin)
    return _tc_c(p2, W2, degp)
